# Initial kernel scaffold; baseline (speedup 1.0000x reference)
import jax
import jax.numpy as jnp
from jax.experimental import pallas as pl


def kernel(input_ids, emb_table, W, b):
    raise NotImplementedError("write your pallas kernel here")


# speedup vs baseline: 79.4549x; 79.4549x over previous
import functools

import jax
import jax.numpy as jnp
from jax import lax
from jax.experimental import pallas as pl
from jax.experimental.pallas import tpu as pltpu
from jax.experimental.pallas import tpu_sc as plsc

B = 16384
L = 200
VOCAB = 100
D = 128
K = 128

NC = 2
NS = 16
LANES = 16
NW = NC * NS
ROWS_PER_W = B // NW
CHUNK = 128
NCHUNK = ROWS_PER_W // CHUNK
GROUPS = CHUNK // LANES

_sc_mesh = plsc.VectorSubcoreMesh(core_axis_name="c", subcore_axis_name="s")


@functools.partial(
    pl.kernel,
    mesh=_sc_mesh,
    out_type=jax.ShapeDtypeStruct((B * D,), jnp.float32),
    scratch_types=[
        pltpu.VMEM((CHUNK * L,), jnp.int32),
        pltpu.VMEM((CHUNK * D,), jnp.float32),
    ],
    compiler_params=pltpu.CompilerParams(needs_layout_passes=False),
)
def _hist(ids_hbm, counts_hbm, ids_v, counts_v):
    wid = lax.axis_index("s") * NC + lax.axis_index("c")
    base = wid * ROWS_PER_W
    lanes = lax.iota(jnp.int32, LANES)
    ones = jnp.ones((LANES,), jnp.float32)
    zeros = jnp.zeros((LANES,), jnp.float32)

    def do_chunk(c, carry):
        row0 = base + c * CHUNK
        pltpu.sync_copy(ids_hbm.at[pl.ds(row0 * L, CHUNK * L)], ids_v)

        def zrow(i, carry2):
            counts_v[pl.ds(i * LANES, LANES)] = zeros
            return carry2

        lax.fori_loop(0, CHUNK * D // LANES, zrow, 0)

        for g in range(GROUPS):
            rows = lanes + g * LANES
            id_base = rows * L
            cnt_base = rows * D

            def body(l, carry3):
                ids16 = plsc.load_gather(ids_v, [id_base + l])
                plsc.addupdate_scatter(counts_v, [cnt_base + ids16], ones)
                return carry3

            lax.fori_loop(0, L, body, 0)

        pltpu.sync_copy(counts_v, counts_hbm.at[pl.ds(row0 * D, CHUNK * D)])
        return carry

    lax.fori_loop(0, NCHUNK, do_chunk, 0)


BT = 256


def _mm_body(emb_ref, w_ref, b_ref, counts_ref, out_ref, m_ref):
    @pl.when(pl.program_id(0) == 0)
    def _():
        m = lax.dot_general(
            emb_ref[...], w_ref[...], (((1,), (1,)), ((), ())),
            preferred_element_type=jnp.float32)
        m_ref[...] = m * (1.0 / L)

    out_ref[...] = (
        jnp.dot(counts_ref[...], m_ref[...],
                preferred_element_type=jnp.float32)
        + b_ref[...])


_matmul = pl.pallas_call(
    _mm_body,
    grid=(B // BT,),
    in_specs=[
        pl.BlockSpec((D, D), lambda i: (0, 0)),
        pl.BlockSpec((K, D), lambda i: (0, 0)),
        pl.BlockSpec((1, K), lambda i: (0, 0)),
        pl.BlockSpec((BT, D), lambda i: (i, 0)),
    ],
    out_specs=pl.BlockSpec((BT, K), lambda i: (i, 0)),
    out_shape=jax.ShapeDtypeStruct((B, K), jnp.float32),
    scratch_shapes=[pltpu.VMEM((D, K), jnp.float32)],
)


def kernel(input_ids, emb_table, W, b):
    ids = input_ids.astype(jnp.int32).reshape(B * L)
    counts = _hist(ids).reshape(B, D)
    emb_pad = jnp.pad(emb_table, ((0, D - VOCAB), (0, 0)))
    return _matmul(emb_pad, W, b.reshape(1, K), counts)


# speedup vs baseline: 99.9370x; 1.2578x over previous
import functools

import jax
import jax.numpy as jnp
from jax import lax
from jax.experimental import pallas as pl
from jax.experimental.pallas import tpu as pltpu
from jax.experimental.pallas import tpu_sc as plsc

B = 16384
L = 200
VOCAB = 100
D = 128
K = 128

NC = 2
NS = 16
LANES = 16
NW = NC * NS
ROWS_PER_W = B // NW
CHUNK = 128
NCHUNK = ROWS_PER_W // CHUNK
GROUPS = CHUNK // LANES

_sc_mesh = plsc.VectorSubcoreMesh(core_axis_name="c", subcore_axis_name="s")


@functools.partial(
    pl.kernel,
    mesh=_sc_mesh,
    out_type=jax.ShapeDtypeStruct((B * D,), jnp.float32),
    scratch_types=[
        pltpu.VMEM((CHUNK * L,), jnp.int32),
        pltpu.VMEM((CHUNK * D,), jnp.float32),
    ],
    compiler_params=pltpu.CompilerParams(needs_layout_passes=False),
)
def _hist(ids_hbm, counts_hbm, ids_v, counts_v):
    wid = lax.axis_index("s") * NC + lax.axis_index("c")
    base = wid * ROWS_PER_W
    lanes = lax.iota(jnp.int32, LANES)
    ones = jnp.ones((LANES,), jnp.float32)
    zeros = jnp.zeros((LANES,), jnp.float32)

    ZU = 8
    HU = 8

    def do_chunk(c, carry):
        row0 = base + c * CHUNK
        pltpu.sync_copy(ids_hbm.at[pl.ds(row0 * L, CHUNK * L)], ids_v)

        def zrow(i, carry2):
            for u in range(ZU):
                counts_v[pl.ds(i * (LANES * ZU) + u * LANES, LANES)] = zeros
            return carry2

        lax.fori_loop(0, CHUNK * D // (LANES * ZU), zrow, 0)

        for g in range(GROUPS):
            rows = lanes + g * LANES
            cnt_base = rows * D

            def body(i, idx):
                for _ in range(HU):
                    ids16 = plsc.load_gather(ids_v, [idx])
                    plsc.addupdate_scatter(counts_v, [cnt_base + ids16], ones)
                    idx = idx + 1
                return idx

            lax.fori_loop(0, L // HU, body, rows * L)

        pltpu.sync_copy(counts_v, counts_hbm.at[pl.ds(row0 * D, CHUNK * D)])
        return carry

    lax.fori_loop(0, NCHUNK, do_chunk, 0)


BT = 1024


def _mm_body(emb_ref, w_ref, b_ref, counts_ref, out_ref, m_ref):
    @pl.when(pl.program_id(0) == 0)
    def _():
        m = lax.dot_general(
            emb_ref[...], w_ref[...], (((1,), (1,)), ((), ())),
            preferred_element_type=jnp.float32)
        m_ref[...] = m * (1.0 / L)

    out_ref[...] = (
        jnp.dot(counts_ref[...], m_ref[...],
                preferred_element_type=jnp.float32)
        + b_ref[...])


_matmul = pl.pallas_call(
    _mm_body,
    grid=(B // BT,),
    in_specs=[
        pl.BlockSpec((D, D), lambda i: (0, 0)),
        pl.BlockSpec((K, D), lambda i: (0, 0)),
        pl.BlockSpec((1, K), lambda i: (0, 0)),
        pl.BlockSpec((BT, D), lambda i: (i, 0)),
    ],
    out_specs=pl.BlockSpec((BT, K), lambda i: (i, 0)),
    out_shape=jax.ShapeDtypeStruct((B, K), jnp.float32),
    scratch_shapes=[pltpu.VMEM((D, K), jnp.float32)],
)


def kernel(input_ids, emb_table, W, b):
    ids = input_ids.astype(jnp.int32).reshape(B * L)
    counts = _hist(ids).reshape(B, D)
    emb_pad = jnp.pad(emb_table, ((0, D - VOCAB), (0, 0)))
    return _matmul(emb_pad, W, b.reshape(1, K), counts)


# speedup vs baseline: 169.4672x; 1.6957x over previous
import functools

import jax
import jax.numpy as jnp
from jax import lax
from jax.experimental import pallas as pl
from jax.experimental.pallas import tpu as pltpu
from jax.experimental.pallas import tpu_sc as plsc

B = 16384
L = 200
VOCAB = 100
D = 128
K = 128

NC = 2
NS = 16
LANES = 16
NW = NC * NS
CHUNK = 128
GROUPS = CHUNK // LANES

_sc_mesh = plsc.VectorSubcoreMesh(core_axis_name="c", subcore_axis_name="s")


def _make_hist(nrows):
    rows_per_w = nrows // NW
    nchunk = rows_per_w // CHUNK

    @functools.partial(
        pl.kernel,
        mesh=_sc_mesh,
        out_type=jax.ShapeDtypeStruct((nrows, D), jnp.float32),
        scratch_types=[
            pltpu.VMEM((CHUNK, L), jnp.int32),
            pltpu.VMEM((CHUNK, L), jnp.int32),
            pltpu.VMEM((CHUNK, D), jnp.float32),
            pltpu.VMEM((CHUNK, D), jnp.float32),
            pltpu.SemaphoreType.DMA,
            pltpu.SemaphoreType.DMA,
            pltpu.SemaphoreType.DMA,
            pltpu.SemaphoreType.DMA,
        ],
        compiler_params=pltpu.CompilerParams(needs_layout_passes=False),
    )
    def _hist(ids_hbm, counts_hbm, ids_v0, ids_v1, counts_v0, counts_v1,
              si0, si1, so0, so1):
        wid = lax.axis_index("s") * NC + lax.axis_index("c")
        base = wid * rows_per_w
        lanes = lax.iota(jnp.int32, LANES)
        ones = jnp.ones((LANES,), jnp.float32)
        zeros = jnp.zeros((LANES,), jnp.float32)
        ids_bufs = [ids_v0, ids_v1]
        counts_bufs = [counts_v0, counts_v1]
        sin = [si0, si1]
        sout = [so0, so1]

        ZU = 8
        HU = 8

        def start_in(c):
            return pltpu.async_copy(
                ids_hbm.at[pl.ds(base + c * CHUNK, CHUNK), :],
                ids_bufs[c & 1], sin[c & 1])

        in_dma = start_in(0)
        out_dma = [None, None]

        for c in range(nchunk):
            bb = c & 1
            in_dma.wait()
            if c + 1 < nchunk:
                in_dma = start_in(c + 1)
            if out_dma[bb] is not None:
                out_dma[bb].wait()

            ids_b = ids_bufs[bb]
            counts_b = counts_bufs[bb]

            def zrow(i, carry2):
                for u in range(ZU):
                    counts_b[i, pl.ds(u * LANES, LANES)] = zeros
                return carry2

            lax.fori_loop(0, CHUNK, zrow, 0)

            for g in range(GROUPS):
                rows = lanes + g * LANES

                def body(i, col):
                    got = []
                    for u in range(HU):
                        t = col + u
                        t = jnp.where(t >= L, t - L, t)
                        got.append(plsc.load_gather(ids_b, [rows, t]))
                    for ids16 in got:
                        plsc.addupdate_scatter(
                            counts_b, [rows, ids16], ones)
                    return col + HU

                lax.fori_loop(0, L // HU, body, lanes)

            out_dma[bb] = pltpu.async_copy(
                counts_b,
                counts_hbm.at[pl.ds(base + c * CHUNK, CHUNK), :],
                sout[bb])

        for h in out_dma:
            if h is not None:
                h.wait()

    return _hist


def _mm_body(emb_ref, w_ref, b_ref, counts_ref, out_ref, m_ref):
    @pl.when(pl.program_id(0) == 0)
    def _():
        m = lax.dot_general(
            emb_ref[...], w_ref[...], (((1,), (1,)), ((), ())),
            preferred_element_type=jnp.float32)
        m_ref[...] = m * (1.0 / L)

    out_ref[...] = (
        jnp.dot(counts_ref[...], m_ref[...],
                preferred_element_type=jnp.float32)
        + b_ref[...])


def _make_matmul(nrows, bt):
    return pl.pallas_call(
        _mm_body,
        grid=(nrows // bt,),
        in_specs=[
            pl.BlockSpec((D, D), lambda i: (0, 0)),
            pl.BlockSpec((K, D), lambda i: (0, 0)),
            pl.BlockSpec((1, K), lambda i: (0, 0)),
            pl.BlockSpec((bt, D), lambda i: (i, 0)),
        ],
        out_specs=pl.BlockSpec((bt, K), lambda i: (i, 0)),
        out_shape=jax.ShapeDtypeStruct((nrows, K), jnp.float32),
        scratch_shapes=[pltpu.VMEM((D, K), jnp.float32)],
    )


B2 = B // 2
_hist_half = _make_hist(B2)
_matmul_half = _make_matmul(B2, 2048)


def kernel(input_ids, emb_table, W, b):
    ids = input_ids.astype(jnp.int32)
    emb_pad = jnp.pad(emb_table, ((0, D - VOCAB), (0, 0)))
    b2 = b.reshape(1, K)
    c1 = _hist_half(ids[:B2])
    c2 = _hist_half(ids[B2:])
    o1 = _matmul_half(emb_pad, W, b2, c1)
    o2 = _matmul_half(emb_pad, W, b2, c2)
    return jnp.concatenate([o1, o2], axis=0)
